# Initial kernel scaffold; baseline (speedup 1.0000x reference)
import functools

import jax
import jax.numpy as jnp
from jax import lax
from jax.experimental import pallas as pl
from jax.experimental.pallas import tpu as pltpu
from jax.experimental.pallas import tpu_sc as plsc

_D = 128
_SENTINEL = 1 << 28


def _pad_idx(idx, mult, fill):
    e = idx.shape[0]
    ep = -(-e // mult) * mult
    if ep == e:
        return idx
    return jnp.concatenate([idx, jnp.full((ep - e,), fill, jnp.int32)])


def _sc_gather(table, idx):
    ep = idx.shape[0]
    nwin = ep // 512
    nwt = -(-nwin // 32)
    idx2 = idx.reshape(ep // 128, 128)
    mesh = plsc.VectorSubcoreMesh(core_axis_name="c", subcore_axis_name="s")

    @functools.partial(
        pl.kernel, mesh=mesh,
        out_type=jax.ShapeDtypeStruct((ep, _D), jnp.float32),
        scratch_types=[
            pltpu.VMEM((4, 128), jnp.int32),
            pltpu.VMEM((512, _D), jnp.float32),
            pltpu.SemaphoreType.DMA,
        ])
    def k(tab, ih, out, idx_v, rows_v, sem):
        wid = lax.axis_index("s") * 2 + lax.axis_index("c")

        def body(w, carry):
            win = w * 32 + wid

            @pl.when(win < nwin)
            def _go():
                pltpu.sync_copy(ih.at[pl.ds(win * 4, 4)], idx_v)
                descs = [
                    pltpu.async_copy(tab.at[idx_v.at[j]],
                                     rows_v.at[pl.ds(j * 128, 128)], sem)
                    for j in range(4)
                ]
                for d in descs:
                    d.wait()
                pltpu.sync_copy(rows_v, out.at[pl.ds(win * 512, 512)])

            return carry

        lax.fori_loop(0, nwt, body, 0)

    return k(table, idx2)


def _sc_scatter_add(upd, idx, n_out, ch):
    ep = upd.shape[0]
    n_chunks = -(-n_out // ch)
    assert n_chunks % 2 == 0 and ep % 512 == 0
    cpc = n_chunks // 2
    spr = ch + 256
    npad = n_chunks * ch
    nwin = ep // 512
    nwt = -(-nwin // 16)
    zr = spr // 16
    cr = ch // 16
    idx2 = idx.reshape(ep // 128, 128)
    zeros = jnp.zeros((spr, _D), jnp.float32)
    mesh = plsc.VectorSubcoreMesh(core_axis_name="c", subcore_axis_name="s")

    @functools.partial(
        pl.kernel, mesh=mesh,
        out_type=jax.ShapeDtypeStruct((npad, _D), jnp.float32),
        scratch_types=[
            pltpu.VMEM((4, 128), jnp.int32),
            pltpu.VMEM((4, 128), jnp.int32),
            pltpu.VMEM((512, _D), jnp.float32),
            pltpu.VMEM_SHARED((spr, _D), jnp.float32),
        ])
    def k(u_hbm, idx_hbm, z_hbm, out_hbm, idx_v, lidx_v, u_v, acc):
        cid = lax.axis_index("c")
        sid = lax.axis_index("s")
        lane = lax.iota(jnp.int32, 16)
        dump = jnp.int32(ch) + sid * 16 + lane
        for c in range(cpc):
            cbase = (2 * c + cid) * ch
            pltpu.sync_copy(z_hbm.at[pl.ds(sid * zr, zr)],
                            acc.at[pl.ds(sid * zr, zr)])
            plsc.subcore_barrier()

            def body(w, carry):
                win = w * 16 + sid

                @pl.when(win < nwin)
                def _go():
                    pltpu.sync_copy(idx_hbm.at[pl.ds(win * 4, 4)], idx_v)
                    pltpu.sync_copy(u_hbm.at[pl.ds(win * 512, 512)], u_v)
                    for j in range(4):
                        for t in range(8):
                            v = idx_v[j, pl.ds(t * 16, 16)]
                            lv = v - cbase
                            ok = (lv >= 0) & (lv < ch)
                            lidx_v[j, pl.ds(t * 16, 16)] = jnp.where(
                                ok, lv, dump)
                    for j in range(4):
                        pltpu.sync_copy(u_v.at[pl.ds(j * 128, 128)],
                                        acc.at[lidx_v.at[j]], add=True)

                return carry

            lax.fori_loop(0, nwt, body, 0)
            plsc.subcore_barrier()
            pltpu.sync_copy(acc.at[pl.ds(sid * cr, cr)],
                            out_hbm.at[pl.ds(cbase + sid * cr, cr)])
            plsc.subcore_barrier()

    return k(upd, idx2, zeros)


def _ln(o, g, b):
    mu = jnp.mean(o, axis=-1, keepdims=True)
    c = o - mu
    var = jnp.mean(c * c, axis=-1, keepdims=True)
    return c * lax.rsqrt(var + 1e-5) * g + b


def _full(shape):
    return pl.BlockSpec(shape, lambda i: (0, 0))


def _edge_mlp(e2d, xs_g, xr_g, p, ns, blk):
    ecnt, de = e2d.shape
    nb = -(-ecnt // blk)
    dh = p["W0"].shape[1]
    w0 = p["W0"]
    w0e = w0[:de]
    w0s = w0[de:de + ns * _D]
    w0r = w0[de + ns * _D:]

    def body(e_ref, xs_ref, xr_ref, w0e_r, w0s_r, w0r_r, b0_r, w1_r, b1_r,
             g_r, bb_r, eu_ref, out_ref):
        e = e_ref[...]
        acc = jnp.dot(e, w0e_r[...], preferred_element_type=jnp.float32)
        xs = xs_ref[...].reshape(blk, ns, _D)
        xr = xr_ref[...].reshape(blk, ns, _D)
        w0s_v = w0s_r[...]
        w0r_v = w0r_r[...]
        for t in range(ns):
            acc += jnp.dot(xs[:, t, :], w0s_v[t * _D:(t + 1) * _D],
                           preferred_element_type=jnp.float32)
            acc += jnp.dot(xr[:, t, :], w0r_v[t * _D:(t + 1) * _D],
                           preferred_element_type=jnp.float32)
        h = jax.nn.relu(acc + b0_r[...])
        o = jnp.dot(h, w1_r[...], preferred_element_type=jnp.float32) \
            + b1_r[...]
        y = _ln(o, g_r[...], bb_r[...])
        eu_ref[...] = y
        out_ref[...] = e + y

    eu, out = pl.pallas_call(
        body,
        grid=(nb,),
        in_specs=[
            pl.BlockSpec((blk, de), lambda i: (i, 0)),
            pl.BlockSpec((ns * blk, _D), lambda i: (i, 0)),
            pl.BlockSpec((ns * blk, _D), lambda i: (i, 0)),
            _full(w0e.shape), _full(w0s.shape), _full(w0r.shape),
            _full((1, dh)), _full(p["W1"].shape), _full((1, de)),
            _full((1, de)), _full((1, de)),
        ],
        out_specs=[
            pl.BlockSpec((blk, de), lambda i: (i, 0)),
            pl.BlockSpec((blk, de), lambda i: (i, 0)),
        ],
        out_shape=[
            jax.ShapeDtypeStruct((nb * blk, de), jnp.float32),
            jax.ShapeDtypeStruct((ecnt, de), jnp.float32),
        ],
    )(e2d, xs_g, xr_g, w0e, w0s, w0r, p["b0"].reshape(1, -1), p["W1"],
      p["b1"].reshape(1, -1), p["gamma"].reshape(1, -1),
      p["beta"].reshape(1, -1))
    return eu, out


def _node_mlp(x, aggrs, p, blk=1024):
    n = x.shape[0]
    nb = -(-n // blk)
    na = len(aggrs)
    dh = p["W0"].shape[1]
    wparts = [p["W0"][t * _D:(t + 1) * _D] for t in range(na + 1)]

    def body(*refs):
        x_ref = refs[0]
        a_refs = refs[1:1 + na]
        w_refs = refs[1 + na:2 + 2 * na]
        b0_r, w1_r, b1_r, g_r, bb_r, out_ref = refs[2 + 2 * na:]
        xv = x_ref[...]
        acc = jnp.dot(xv, w_refs[0][...], preferred_element_type=jnp.float32)
        for t in range(na):
            acc += jnp.dot(a_refs[t][...], w_refs[t + 1][...],
                           preferred_element_type=jnp.float32)
        h = jax.nn.relu(acc + b0_r[...])
        o = jnp.dot(h, w1_r[...], preferred_element_type=jnp.float32) \
            + b1_r[...]
        out_ref[...] = xv + _ln(o, g_r[...], bb_r[...])

    out = pl.pallas_call(
        body,
        grid=(nb,),
        in_specs=(
            [pl.BlockSpec((blk, _D), lambda i: (i, 0))] * (1 + na)
            + [_full((_D, dh))] * (1 + na)
            + [_full((1, dh)), _full(p["W1"].shape), _full((1, _D)),
               _full((1, _D)), _full((1, _D))]
        ),
        out_specs=pl.BlockSpec((blk, _D), lambda i: (i, 0)),
        out_shape=jax.ShapeDtypeStruct((n, _D), jnp.float32),
    )(x, *aggrs, *wparts, p["b0"].reshape(1, -1), p["W1"],
      p["b1"].reshape(1, -1), p["gamma"].reshape(1, -1),
      p["beta"].reshape(1, -1))
    return out


def kernel(mesh_n, obj_n, mm_index, mo_index, om_index, ff_index,
           e_mm, e_mo, e_om, e_ff, params):
    s, r = 0, 1
    nm = mesh_n.shape[0]
    no = obj_n.shape[0]
    eff = e_ff.shape[0]

    def gidx(a):
        return _pad_idx(a.astype(jnp.int32), 512, 0)

    mm_s = _sc_gather(mesh_n, gidx(mm_index[s]))
    mm_r = _sc_gather(mesh_n, gidx(mm_index[r]))
    mo_s = _sc_gather(mesh_n, gidx(mo_index[s]))
    mo_r = _sc_gather(obj_n, gidx(mo_index[r]))
    om_s = _sc_gather(obj_n, gidx(om_index[s]))
    om_r = _sc_gather(mesh_n, gidx(om_index[r]))
    ff_s = _sc_gather(mesh_n, gidx(ff_index[s].reshape(-1)))
    ff_r = _sc_gather(mesh_n, gidx(ff_index[r].reshape(-1)))

    eu_mm, out_mm = _edge_mlp(e_mm, mm_s, mm_r, params["mm"], 1, 1024)
    eu_mo, out_mo = _edge_mlp(e_mo, mo_s, mo_r, params["mo"], 1, 1024)
    eu_om, out_om = _edge_mlp(e_om, om_s, om_r, params["om"], 1, 1024)
    eu_ff, out_ff = _edge_mlp(e_ff.reshape(eff, 3 * _D), ff_s, ff_r,
                              params["ff"], 3, 512)

    def sidx(a, ep):
        a = a.astype(jnp.int32)
        return jnp.concatenate(
            [a, jnp.full((ep - a.shape[0],), _SENTINEL, jnp.int32)])

    aggr_mm = _sc_scatter_add(eu_mm, sidx(mm_index[r], eu_mm.shape[0]),
                              nm, 12800)
    aggr_mo = _sc_scatter_add(eu_mo, sidx(mo_index[r], eu_mo.shape[0]),
                              no, 5120)
    aggr_om = _sc_scatter_add(eu_om, sidx(om_index[r], eu_om.shape[0]),
                              nm, 12800)
    eu_ff_rows = eu_ff.reshape(-1, _D)
    aggr_ff = _sc_scatter_add(eu_ff_rows,
                              sidx(ff_index[r].reshape(-1),
                                   eu_ff_rows.shape[0]),
                              nm, 12800)

    obj_out = _node_mlp(obj_n, [aggr_mo], params["obj"])
    mesh_out = _node_mlp(mesh_n, [aggr_om, aggr_mm, aggr_ff], params["mesh"])

    return (mesh_out, obj_out, out_mm, out_mo, out_om,
            out_ff.reshape(eff, 3, _D))


# speedup vs baseline: 3.6716x; 3.6716x over previous
import functools

import jax
import jax.numpy as jnp
from jax import lax
from jax.experimental import pallas as pl
from jax.experimental.pallas import tpu as pltpu
from jax.experimental.pallas import tpu_sc as plsc

_D = 128
_SENTINEL = 1 << 28


def _pad_idx(idx, mult, fill):
    e = idx.shape[0]
    ep = -(-e // mult) * mult
    if ep == e:
        return idx
    return jnp.concatenate([idx, jnp.full((ep - e,), fill, jnp.int32)])


def _sc_gather(table, idx):
    ep = idx.shape[0]
    nwin = ep // 512
    nwt = -(-nwin // 32)
    idx2 = idx.reshape(ep // 128, 128)
    mesh = plsc.VectorSubcoreMesh(core_axis_name="c", subcore_axis_name="s")

    @functools.partial(
        pl.kernel, mesh=mesh,
        out_type=jax.ShapeDtypeStruct((ep, _D), jnp.float32),
        scratch_types=[
            pltpu.VMEM((4, 128), jnp.int32),
            pltpu.VMEM((512, _D), jnp.float32),
            pltpu.SemaphoreType.DMA,
        ])
    def k(tab, ih, out, idx_v, rows_v, sem):
        wid = lax.axis_index("s") * 2 + lax.axis_index("c")

        def body(w, carry):
            win = w * 32 + wid

            @pl.when(win < nwin)
            def _go():
                pltpu.sync_copy(ih.at[pl.ds(win * 4, 4)], idx_v)
                descs = [
                    pltpu.async_copy(tab.at[idx_v.at[j]],
                                     rows_v.at[pl.ds(j * 128, 128)], sem)
                    for j in range(4)
                ]
                for d in descs:
                    d.wait()
                pltpu.sync_copy(rows_v, out.at[pl.ds(win * 512, 512)])

            return carry

        lax.fori_loop(0, nwt, body, 0)

    return k(table, idx2)


def _sc_scatter_add(upd, idx, n_out, ch):
    ep = upd.shape[0]
    n_chunks = -(-n_out // ch)
    assert n_chunks % 2 == 0 and ep % 512 == 0
    cpc = n_chunks // 2
    spr = ch + 256
    npad = n_chunks * ch
    nwin = ep // 512
    nwt = -(-nwin // 16)
    zr = spr // 16
    cr = ch // 16
    idx2 = idx.reshape(ep // 128, 128)
    zeros = jnp.zeros((spr, _D), jnp.float32)
    mesh = plsc.VectorSubcoreMesh(core_axis_name="c", subcore_axis_name="s")

    @functools.partial(
        pl.kernel, mesh=mesh,
        out_type=jax.ShapeDtypeStruct((npad, _D), jnp.float32),
        scratch_types=[
            pltpu.VMEM((4, 128), jnp.int32),
            pltpu.VMEM((4, 128), jnp.int32),
            pltpu.VMEM((512, _D), jnp.float32),
            pltpu.VMEM_SHARED((spr, _D), jnp.float32),
        ])
    def k(u_hbm, idx_hbm, z_hbm, out_hbm, idx_v, lidx_v, u_v, acc):
        cid = lax.axis_index("c")
        sid = lax.axis_index("s")
        lane = lax.iota(jnp.int32, 16)
        dump = jnp.int32(ch) + sid * 16 + lane
        for c in range(cpc):
            cbase = (2 * c + cid) * ch
            pltpu.sync_copy(z_hbm.at[pl.ds(sid * zr, zr)],
                            acc.at[pl.ds(sid * zr, zr)])
            plsc.subcore_barrier()

            def body(w, carry):
                win = w * 16 + sid

                @pl.when(win < nwin)
                def _go():
                    pltpu.sync_copy(idx_hbm.at[pl.ds(win * 4, 4)], idx_v)
                    pltpu.sync_copy(u_hbm.at[pl.ds(win * 512, 512)], u_v)
                    for j in range(4):
                        for t in range(8):
                            v = idx_v[j, pl.ds(t * 16, 16)]
                            lv = v - cbase
                            ok = (lv >= 0) & (lv < ch)
                            lidx_v[j, pl.ds(t * 16, 16)] = jnp.where(
                                ok, lv, dump)
                    for j in range(4):
                        pltpu.sync_copy(u_v.at[pl.ds(j * 128, 128)],
                                        acc.at[lidx_v.at[j]], add=True)

                return carry

            lax.fori_loop(0, nwt, body, 0)
            plsc.subcore_barrier()
            pltpu.sync_copy(acc.at[pl.ds(sid * cr, cr)],
                            out_hbm.at[pl.ds(cbase + sid * cr, cr)])
            plsc.subcore_barrier()

    return k(upd, idx2, zeros)


def _ln(o, g, b):
    mu = jnp.mean(o, axis=-1, keepdims=True)
    c = o - mu
    var = jnp.mean(c * c, axis=-1, keepdims=True)
    return c * lax.rsqrt(var + 1e-5) * g + b


def _full(shape):
    return pl.BlockSpec(shape, lambda i: (0, 0))


def _edge_mlp(e2d, xs_g, xr_g, p, ns, blk):
    ecnt, de = e2d.shape
    nb = -(-ecnt // blk)
    dh = p["W0"].shape[1]
    w4 = p["W0"].reshape(ns, 3, _D, dh)
    w0e = w4[:, 0].reshape(ns * _D, dh)
    w0s = w4[:, 1].reshape(ns * _D, dh)
    w0r = w4[:, 2].reshape(ns * _D, dh)

    def body(e_ref, xs_ref, xr_ref, w0e_r, w0s_r, w0r_r, b0_r, w1_r, b1_r,
             g_r, bb_r, eu_ref, out_ref):
        e = e_ref[...]
        e3 = e.reshape(blk, ns, _D)
        xs = xs_ref[...].reshape(blk, ns, _D)
        xr = xr_ref[...].reshape(blk, ns, _D)
        w0e_v = w0e_r[...]
        w0s_v = w0s_r[...]
        w0r_v = w0r_r[...]
        acc = jnp.zeros((blk, dh), jnp.float32)
        for t in range(ns):
            acc += jnp.dot(e3[:, t, :], w0e_v[t * _D:(t + 1) * _D],
                           preferred_element_type=jnp.float32)
            acc += jnp.dot(xs[:, t, :], w0s_v[t * _D:(t + 1) * _D],
                           preferred_element_type=jnp.float32)
            acc += jnp.dot(xr[:, t, :], w0r_v[t * _D:(t + 1) * _D],
                           preferred_element_type=jnp.float32)
        h = jax.nn.relu(acc + b0_r[...])
        o = jnp.dot(h, w1_r[...], preferred_element_type=jnp.float32) \
            + b1_r[...]
        y = _ln(o, g_r[...], bb_r[...])
        eu_ref[...] = y
        out_ref[...] = e + y

    eu, out = pl.pallas_call(
        body,
        grid=(nb,),
        in_specs=[
            pl.BlockSpec((blk, de), lambda i: (i, 0)),
            pl.BlockSpec((ns * blk, _D), lambda i: (i, 0)),
            pl.BlockSpec((ns * blk, _D), lambda i: (i, 0)),
            _full(w0e.shape), _full(w0s.shape), _full(w0r.shape),
            _full((1, dh)), _full(p["W1"].shape), _full((1, de)),
            _full((1, de)), _full((1, de)),
        ],
        out_specs=[
            pl.BlockSpec((blk, de), lambda i: (i, 0)),
            pl.BlockSpec((blk, de), lambda i: (i, 0)),
        ],
        out_shape=[
            jax.ShapeDtypeStruct((nb * blk, de), jnp.float32),
            jax.ShapeDtypeStruct((ecnt, de), jnp.float32),
        ],
    )(e2d, xs_g, xr_g, w0e, w0s, w0r, p["b0"].reshape(1, -1), p["W1"],
      p["b1"].reshape(1, -1), p["gamma"].reshape(1, -1),
      p["beta"].reshape(1, -1))
    return eu, out


def _node_mlp(x, aggrs, p, blk=1024):
    n = x.shape[0]
    nb = -(-n // blk)
    na = len(aggrs)
    dh = p["W0"].shape[1]
    wparts = [p["W0"][t * _D:(t + 1) * _D] for t in range(na + 1)]

    def body(*refs):
        x_ref = refs[0]
        a_refs = refs[1:1 + na]
        w_refs = refs[1 + na:2 + 2 * na]
        b0_r, w1_r, b1_r, g_r, bb_r, out_ref = refs[2 + 2 * na:]
        xv = x_ref[...]
        acc = jnp.dot(xv, w_refs[0][...], preferred_element_type=jnp.float32)
        for t in range(na):
            acc += jnp.dot(a_refs[t][...], w_refs[t + 1][...],
                           preferred_element_type=jnp.float32)
        h = jax.nn.relu(acc + b0_r[...])
        o = jnp.dot(h, w1_r[...], preferred_element_type=jnp.float32) \
            + b1_r[...]
        out_ref[...] = xv + _ln(o, g_r[...], bb_r[...])

    out = pl.pallas_call(
        body,
        grid=(nb,),
        in_specs=(
            [pl.BlockSpec((blk, _D), lambda i: (i, 0))] * (1 + na)
            + [_full((_D, dh))] * (1 + na)
            + [_full((1, dh)), _full(p["W1"].shape), _full((1, _D)),
               _full((1, _D)), _full((1, _D))]
        ),
        out_specs=pl.BlockSpec((blk, _D), lambda i: (i, 0)),
        out_shape=jax.ShapeDtypeStruct((n, _D), jnp.float32),
    )(x, *aggrs, *wparts, p["b0"].reshape(1, -1), p["W1"],
      p["b1"].reshape(1, -1), p["gamma"].reshape(1, -1),
      p["beta"].reshape(1, -1))
    return out


def kernel(mesh_n, obj_n, mm_index, mo_index, om_index, ff_index,
           e_mm, e_mo, e_om, e_ff, params):
    s, r = 0, 1
    nm = mesh_n.shape[0]
    no = obj_n.shape[0]
    eff = e_ff.shape[0]

    def gidx(a):
        return _pad_idx(a.astype(jnp.int32), 512, 0)

    _DBG_GATHER = False
    _DBG_SCATTER = False
    if _DBG_GATHER:
        def _gath(table, idx):
            return jnp.take(table, idx, axis=0)
    else:
        _gath = _sc_gather
    if _DBG_SCATTER:
        def _scat(upd, idx, n_out, ch):
            n_chunks = -(-n_out // ch)
            return jax.ops.segment_sum(
                upd, jnp.minimum(idx, n_chunks * ch - 1),
                num_segments=n_chunks * ch)
    else:
        _scat = _sc_scatter_add

    mm_s = _gath(mesh_n, gidx(mm_index[s]))
    mm_r = _gath(mesh_n, gidx(mm_index[r]))
    mo_s = _gath(mesh_n, gidx(mo_index[s]))
    mo_r = _gath(obj_n, gidx(mo_index[r]))
    om_s = _gath(obj_n, gidx(om_index[s]))
    om_r = _gath(mesh_n, gidx(om_index[r]))
    ff_s = _gath(mesh_n, gidx(ff_index[s].reshape(-1)))
    ff_r = _gath(mesh_n, gidx(ff_index[r].reshape(-1)))

    eu_mm, out_mm = _edge_mlp(e_mm, mm_s, mm_r, params["mm"], 1, 1024)
    eu_mo, out_mo = _edge_mlp(e_mo, mo_s, mo_r, params["mo"], 1, 1024)
    eu_om, out_om = _edge_mlp(e_om, om_s, om_r, params["om"], 1, 1024)
    eu_ff, out_ff = _edge_mlp(e_ff.reshape(eff, 3 * _D), ff_s, ff_r,
                              params["ff"], 3, 512)

    def sidx(a, ep):
        a = a.astype(jnp.int32)
        return jnp.concatenate(
            [a, jnp.full((ep - a.shape[0],), _SENTINEL, jnp.int32)])

    aggr_mm = _scat(eu_mm, sidx(mm_index[r], eu_mm.shape[0]), nm, 6400)
    aggr_mo = _scat(eu_mo, sidx(mo_index[r], eu_mo.shape[0]), no, 5120)
    aggr_om = _scat(eu_om, sidx(om_index[r], eu_om.shape[0]), nm, 6400)
    eu_ff_rows = eu_ff.reshape(-1, _D)
    aggr_ff = _scat(eu_ff_rows,
                    sidx(ff_index[r].reshape(-1), eu_ff_rows.shape[0]),
                    nm, 6400)

    obj_out = _node_mlp(obj_n, [aggr_mo], params["obj"])
    mesh_out = _node_mlp(mesh_n, [aggr_om, aggr_mm, aggr_ff], params["mesh"])

    return (mesh_out, obj_out, out_mm, out_mo, out_om,
            out_ff.reshape(eff, 3, _D))


# speedup vs baseline: 4.5607x; 1.2422x over previous
import functools

import jax
import jax.numpy as jnp
from jax import lax
from jax.experimental import pallas as pl
from jax.experimental.pallas import tpu as pltpu
from jax.experimental.pallas import tpu_sc as plsc

_D = 128
_SENTINEL = 1 << 28


def _pad_idx(idx, mult, fill):
    e = idx.shape[0]
    ep = -(-e // mult) * mult
    if ep == e:
        return idx
    return jnp.concatenate([idx, jnp.full((ep - e,), fill, jnp.int32)])


def _sc_gather(table, idx):
    ep = idx.shape[0]
    nwin = ep // 512
    nwt = -(-nwin // 32)
    idx2 = idx.reshape(ep // 128, 128)
    mesh = plsc.VectorSubcoreMesh(core_axis_name="c", subcore_axis_name="s")

    @functools.partial(
        pl.kernel, mesh=mesh,
        out_type=jax.ShapeDtypeStruct((ep, _D), jnp.float32),
        scratch_types=[
            pltpu.VMEM((4, 128), jnp.int32),
            pltpu.VMEM((512, _D), jnp.float32),
            pltpu.SemaphoreType.DMA,
        ])
    def k(tab, ih, out, idx_v, rows_v, sem):
        wid = lax.axis_index("s") * 2 + lax.axis_index("c")

        def body(w, carry):
            win = w * 32 + wid

            @pl.when(win < nwin)
            def _go():
                pltpu.sync_copy(ih.at[pl.ds(win * 4, 4)], idx_v)
                descs = [
                    pltpu.async_copy(tab.at[idx_v.at[j]],
                                     rows_v.at[pl.ds(j * 128, 128)], sem)
                    for j in range(4)
                ]
                for d in descs:
                    d.wait()
                pltpu.sync_copy(rows_v, out.at[pl.ds(win * 512, 512)])

            return carry

        lax.fori_loop(0, nwt, body, 0)

    return k(table, idx2)


def _sc_scatter_add(upd, idx, n_out, ch):
    ep = upd.shape[0]
    n_chunks = -(-n_out // ch)
    assert n_chunks % 2 == 0 and ep % 512 == 0
    cpc = n_chunks // 2
    spr = ch + 256
    npad = n_chunks * ch
    nwin = ep // 256
    nwt = -(-nwin // 16)
    nwt2 = -(-nwt // 2)
    zr = spr // 16
    cr = ch // 16
    idx2 = idx.reshape(ep // 128, 128)
    zeros = jnp.zeros((spr, _D), jnp.float32)
    mesh = plsc.VectorSubcoreMesh(core_axis_name="c", subcore_axis_name="s")

    @functools.partial(
        pl.kernel, mesh=mesh,
        out_type=jax.ShapeDtypeStruct((npad, _D), jnp.float32),
        scratch_types=[
            pltpu.VMEM((4, 128), jnp.int32),
            pltpu.VMEM((4, 128), jnp.int32),
            pltpu.VMEM((512, _D), jnp.float32),
            pltpu.VMEM_SHARED((spr, _D), jnp.float32),
            pltpu.SemaphoreType.DMA,
            pltpu.SemaphoreType.DMA,
            pltpu.SemaphoreType.DMA,
        ])
    def k(u_hbm, idx_hbm, z_hbm, out_hbm, idx_v, lidx_v, u_v, acc,
          isem, usem, ssem):
        cid = lax.axis_index("c")
        sid = lax.axis_index("s")
        lane = lax.iota(jnp.int32, 16)
        dump = jnp.int32(ch) + sid * 16 + lane

        def fire_loads(win, p):
            pltpu.async_copy(idx_hbm.at[pl.ds(win * 2, 2)],
                             idx_v.at[pl.ds(p * 2, 2)], isem)
            pltpu.async_copy(u_hbm.at[pl.ds(win * 256, 256)],
                             u_v.at[pl.ds(p * 256, 256)], usem)

        def iwait():
            pltpu.make_async_copy(idx_hbm.at[pl.ds(0, 2)],
                                  idx_v.at[pl.ds(0, 2)], isem).wait()

        def uwait():
            pltpu.make_async_copy(u_hbm.at[pl.ds(0, 256)],
                                  u_v.at[pl.ds(0, 256)], usem).wait()

        def swait():
            pltpu.make_async_copy(u_hbm.at[pl.ds(0, 128)],
                                  u_v.at[pl.ds(0, 128)], ssem).wait()

        def sub_window(w, p, cbase, drain_pred):
            if drain_pred is None:
                swait()
                swait()
            else:
                @pl.when(drain_pred)
                def _():
                    swait()
                    swait()

            @pl.when((w + 1) * 16 + sid < nwin)
            def _():
                fire_loads((w + 1) * 16 + sid, 1 - p)
            iwait()
            uwait()
            for j in range(2):
                for t in range(8):
                    v = idx_v[p * 2 + j, pl.ds(t * 16, 16)]
                    lv = v - cbase
                    ok = (lv >= 0) & (lv < ch)
                    lidx_v[p * 2 + j, pl.ds(t * 16, 16)] = jnp.where(
                        ok, lv, dump)
            for j in range(2):
                pltpu.async_copy(u_v.at[pl.ds(p * 256 + j * 128, 128)],
                                 acc.at[lidx_v.at[p * 2 + j]], ssem,
                                 add=True)

        for c in range(cpc):
            cbase = (2 * c + cid) * ch
            pltpu.sync_copy(z_hbm.at[pl.ds(sid * zr, zr)],
                            acc.at[pl.ds(sid * zr, zr)])

            @pl.when(sid < nwin)
            def _():
                fire_loads(sid, 0)
            plsc.subcore_barrier()

            def body(w2, carry):
                wa = 2 * w2

                @pl.when(wa * 16 + sid < nwin)
                def _():
                    sub_window(wa, 0, cbase, w2 >= 1)

                @pl.when((wa + 1) * 16 + sid < nwin)
                def _():
                    sub_window(wa + 1, 1, cbase, None)
                return carry

            lax.fori_loop(0, nwt2, body, 0)

            @pl.when(sid < nwin)
            def _():
                swait()
                swait()
            plsc.subcore_barrier()
            pltpu.sync_copy(acc.at[pl.ds(sid * cr, cr)],
                            out_hbm.at[pl.ds(cbase + sid * cr, cr)])
            plsc.subcore_barrier()

    return k(upd, idx2, zeros)


def _ln(o, g, b):
    mu = jnp.mean(o, axis=-1, keepdims=True)
    c = o - mu
    var = jnp.mean(c * c, axis=-1, keepdims=True)
    return c * lax.rsqrt(var + 1e-5) * g + b


def _full(shape):
    return pl.BlockSpec(shape, lambda i: (0, 0))


def _edge_mlp(e2d, xs_g, xr_g, p, ns, blk):
    ecnt, de = e2d.shape
    nb = -(-ecnt // blk)
    dh = p["W0"].shape[1]
    w4 = p["W0"].reshape(ns, 3, _D, dh)
    w0e = w4[:, 0].reshape(ns * _D, dh)
    w0s = w4[:, 1].reshape(ns * _D, dh)
    w0r = w4[:, 2].reshape(ns * _D, dh)

    def body(e_ref, xs_ref, xr_ref, w0e_r, w0s_r, w0r_r, b0_r, w1_r, b1_r,
             g_r, bb_r, eu_ref, out_ref):
        e = e_ref[...]
        e3 = e.reshape(blk, ns, _D)
        xs = xs_ref[...].reshape(blk, ns, _D)
        xr = xr_ref[...].reshape(blk, ns, _D)
        w0e_v = w0e_r[...]
        w0s_v = w0s_r[...]
        w0r_v = w0r_r[...]
        acc = jnp.zeros((blk, dh), jnp.float32)
        for t in range(ns):
            acc += jnp.dot(e3[:, t, :], w0e_v[t * _D:(t + 1) * _D],
                           preferred_element_type=jnp.float32)
            acc += jnp.dot(xs[:, t, :], w0s_v[t * _D:(t + 1) * _D],
                           preferred_element_type=jnp.float32)
            acc += jnp.dot(xr[:, t, :], w0r_v[t * _D:(t + 1) * _D],
                           preferred_element_type=jnp.float32)
        h = jax.nn.relu(acc + b0_r[...])
        o = jnp.dot(h, w1_r[...], preferred_element_type=jnp.float32) \
            + b1_r[...]
        y = _ln(o, g_r[...], bb_r[...])
        eu_ref[...] = y
        out_ref[...] = e + y

    eu, out = pl.pallas_call(
        body,
        grid=(nb,),
        in_specs=[
            pl.BlockSpec((blk, de), lambda i: (i, 0)),
            pl.BlockSpec((ns * blk, _D), lambda i: (i, 0)),
            pl.BlockSpec((ns * blk, _D), lambda i: (i, 0)),
            _full(w0e.shape), _full(w0s.shape), _full(w0r.shape),
            _full((1, dh)), _full(p["W1"].shape), _full((1, de)),
            _full((1, de)), _full((1, de)),
        ],
        out_specs=[
            pl.BlockSpec((blk, de), lambda i: (i, 0)),
            pl.BlockSpec((blk, de), lambda i: (i, 0)),
        ],
        out_shape=[
            jax.ShapeDtypeStruct((nb * blk, de), jnp.float32),
            jax.ShapeDtypeStruct((ecnt, de), jnp.float32),
        ],
    )(e2d, xs_g, xr_g, w0e, w0s, w0r, p["b0"].reshape(1, -1), p["W1"],
      p["b1"].reshape(1, -1), p["gamma"].reshape(1, -1),
      p["beta"].reshape(1, -1))
    return eu, out


def _node_mlp(x, aggrs, p, blk=1024):
    n = x.shape[0]
    nb = -(-n // blk)
    na = len(aggrs)
    dh = p["W0"].shape[1]
    wparts = [p["W0"][t * _D:(t + 1) * _D] for t in range(na + 1)]

    def body(*refs):
        x_ref = refs[0]
        a_refs = refs[1:1 + na]
        w_refs = refs[1 + na:2 + 2 * na]
        b0_r, w1_r, b1_r, g_r, bb_r, out_ref = refs[2 + 2 * na:]
        xv = x_ref[...]
        acc = jnp.dot(xv, w_refs[0][...], preferred_element_type=jnp.float32)
        for t in range(na):
            acc += jnp.dot(a_refs[t][...], w_refs[t + 1][...],
                           preferred_element_type=jnp.float32)
        h = jax.nn.relu(acc + b0_r[...])
        o = jnp.dot(h, w1_r[...], preferred_element_type=jnp.float32) \
            + b1_r[...]
        out_ref[...] = xv + _ln(o, g_r[...], bb_r[...])

    out = pl.pallas_call(
        body,
        grid=(nb,),
        in_specs=(
            [pl.BlockSpec((blk, _D), lambda i: (i, 0))] * (1 + na)
            + [_full((_D, dh))] * (1 + na)
            + [_full((1, dh)), _full(p["W1"].shape), _full((1, _D)),
               _full((1, _D)), _full((1, _D))]
        ),
        out_specs=pl.BlockSpec((blk, _D), lambda i: (i, 0)),
        out_shape=jax.ShapeDtypeStruct((n, _D), jnp.float32),
    )(x, *aggrs, *wparts, p["b0"].reshape(1, -1), p["W1"],
      p["b1"].reshape(1, -1), p["gamma"].reshape(1, -1),
      p["beta"].reshape(1, -1))
    return out


def kernel(mesh_n, obj_n, mm_index, mo_index, om_index, ff_index,
           e_mm, e_mo, e_om, e_ff, params):
    s, r = 0, 1
    nm = mesh_n.shape[0]
    no = obj_n.shape[0]
    eff = e_ff.shape[0]

    def gidx(a):
        return _pad_idx(a.astype(jnp.int32), 512, 0)

    _DBG_GATHER = False
    _DBG_SCATTER = False
    if _DBG_GATHER:
        def _gath(table, idx):
            return jnp.take(table, idx, axis=0)
    else:
        _gath = _sc_gather
    if _DBG_SCATTER:
        def _scat(upd, idx, n_out, ch):
            n_chunks = -(-n_out // ch)
            return jax.ops.segment_sum(
                upd, jnp.minimum(idx, n_chunks * ch - 1),
                num_segments=n_chunks * ch)
    else:
        _scat = _sc_scatter_add

    mm_s = _gath(mesh_n, gidx(mm_index[s]))
    mm_r = _gath(mesh_n, gidx(mm_index[r]))
    mo_s = _gath(mesh_n, gidx(mo_index[s]))
    mo_r = _gath(obj_n, gidx(mo_index[r]))
    om_s = _gath(obj_n, gidx(om_index[s]))
    om_r = _gath(mesh_n, gidx(om_index[r]))
    ff_s = _gath(mesh_n, gidx(ff_index[s].reshape(-1)))
    ff_r = _gath(mesh_n, gidx(ff_index[r].reshape(-1)))

    eu_mm, out_mm = _edge_mlp(e_mm, mm_s, mm_r, params["mm"], 1, 1024)
    eu_mo, out_mo = _edge_mlp(e_mo, mo_s, mo_r, params["mo"], 1, 1024)
    eu_om, out_om = _edge_mlp(e_om, om_s, om_r, params["om"], 1, 1024)
    eu_ff, out_ff = _edge_mlp(e_ff.reshape(eff, 3 * _D), ff_s, ff_r,
                              params["ff"], 3, 512)

    def sidx(a, ep):
        a = a.astype(jnp.int32)
        return jnp.concatenate(
            [a, jnp.full((ep - a.shape[0],), _SENTINEL, jnp.int32)])

    aggr_mm = _scat(eu_mm, sidx(mm_index[r], eu_mm.shape[0]), nm, 6400)
    aggr_mo = _scat(eu_mo, sidx(mo_index[r], eu_mo.shape[0]), no, 5120)
    aggr_om = _scat(eu_om, sidx(om_index[r], eu_om.shape[0]), nm, 6400)
    eu_ff_rows = eu_ff.reshape(-1, _D)
    aggr_ff = _scat(eu_ff_rows,
                    sidx(ff_index[r].reshape(-1), eu_ff_rows.shape[0]),
                    nm, 6400)

    obj_out = _node_mlp(obj_n, [aggr_mo], params["obj"])
    mesh_out = _node_mlp(mesh_n, [aggr_om, aggr_mm, aggr_ff], params["mesh"])

    return (mesh_out, obj_out, out_mm, out_mo, out_om,
            out_ff.reshape(eff, 3, _D))


# speedup vs baseline: 4.5766x; 1.0035x over previous
import functools

import jax
import jax.numpy as jnp
from jax import lax
from jax.experimental import pallas as pl
from jax.experimental.pallas import tpu as pltpu
from jax.experimental.pallas import tpu_sc as plsc

_D = 128
_SENTINEL = 1 << 28


def _pad_idx(idx, mult, fill):
    e = idx.shape[0]
    ep = -(-e // mult) * mult
    if ep == e:
        return idx
    return jnp.concatenate([idx, jnp.full((ep - e,), fill, jnp.int32)])


def _sc_gather(table, idx):
    ep = idx.shape[0]
    nwin = ep // 256
    nwt = -(-nwin // 32)
    nwt2 = -(-nwt // 2)
    idx2 = idx.reshape(ep // 128, 128)
    mesh = plsc.VectorSubcoreMesh(core_axis_name="c", subcore_axis_name="s")

    @functools.partial(
        pl.kernel, mesh=mesh,
        out_type=jax.ShapeDtypeStruct((ep, _D), jnp.float32),
        scratch_types=[
            pltpu.VMEM((4, 128), jnp.int32),
            pltpu.VMEM((512, _D), jnp.float32),
            pltpu.SemaphoreType.DMA,
            pltpu.SemaphoreType.DMA,
            pltpu.SemaphoreType.DMA,
        ])
    def k(tab, ih, out, idx_v, rows_v, isem, gsem, osem):
        wid = lax.axis_index("s") * 2 + lax.axis_index("c")

        def iwait():
            pltpu.make_async_copy(ih.at[pl.ds(0, 2)],
                                  idx_v.at[pl.ds(0, 2)], isem).wait()

        def owait():
            pltpu.make_async_copy(tab.at[pl.ds(0, 256)],
                                  rows_v.at[pl.ds(0, 256)], osem).wait()

        def sub_window(w, p, drain_pred):
            if drain_pred is None:
                owait()
            else:
                @pl.when(drain_pred)
                def _():
                    owait()

            win = w * 32 + wid

            @pl.when(win + 32 < nwin)
            def _():
                pltpu.async_copy(ih.at[pl.ds((win + 32) * 2, 2)],
                                 idx_v.at[pl.ds((1 - p) * 2, 2)], isem)
            iwait()
            descs = [
                pltpu.async_copy(
                    tab.at[idx_v.at[p * 2 + j]],
                    rows_v.at[pl.ds(p * 256 + j * 128, 128)], gsem)
                for j in range(2)
            ]
            for d in descs:
                d.wait()
            pltpu.async_copy(rows_v.at[pl.ds(p * 256, 256)],
                             out.at[pl.ds(win * 256, 256)], osem)

        @pl.when(wid < nwin)
        def _():
            pltpu.async_copy(ih.at[pl.ds(wid * 2, 2)],
                             idx_v.at[pl.ds(0, 2)], isem)

        def body(w2, carry):
            wa = 2 * w2

            @pl.when(wa * 32 + wid < nwin)
            def _():
                sub_window(wa, 0, w2 >= 1)

            @pl.when((wa + 1) * 32 + wid < nwin)
            def _():
                sub_window(wa + 1, 1, None)
            return carry

        lax.fori_loop(0, nwt2, body, 0)

        @pl.when(wid < nwin)
        def _():
            owait()

    return k(table, idx2)


def _sc_scatter_add(upd, idx, n_out, ch):
    ep = upd.shape[0]
    n_chunks = -(-n_out // ch)
    assert n_chunks % 2 == 0 and ep % 512 == 0
    cpc = n_chunks // 2
    spr = ch + 256
    npad = n_chunks * ch
    nwin = ep // 256
    nwt = -(-nwin // 16)
    nwt2 = -(-nwt // 2)
    zr = spr // 16
    cr = ch // 16
    idx2 = idx.reshape(ep // 128, 128)
    zeros = jnp.zeros((spr, _D), jnp.float32)
    mesh = plsc.VectorSubcoreMesh(core_axis_name="c", subcore_axis_name="s")

    @functools.partial(
        pl.kernel, mesh=mesh,
        out_type=jax.ShapeDtypeStruct((npad, _D), jnp.float32),
        scratch_types=[
            pltpu.VMEM((4, 128), jnp.int32),
            pltpu.VMEM((4, 128), jnp.int32),
            pltpu.VMEM((512, _D), jnp.float32),
            pltpu.VMEM_SHARED((spr, _D), jnp.float32),
            pltpu.SemaphoreType.DMA,
            pltpu.SemaphoreType.DMA,
            pltpu.SemaphoreType.DMA,
        ])
    def k(u_hbm, idx_hbm, z_hbm, out_hbm, idx_v, lidx_v, u_v, acc,
          isem, usem, ssem):
        cid = lax.axis_index("c")
        sid = lax.axis_index("s")
        lane = lax.iota(jnp.int32, 16)
        dump = jnp.int32(ch) + sid * 16 + lane

        def fire_loads(win, p):
            pltpu.async_copy(idx_hbm.at[pl.ds(win * 2, 2)],
                             idx_v.at[pl.ds(p * 2, 2)], isem)
            pltpu.async_copy(u_hbm.at[pl.ds(win * 256, 256)],
                             u_v.at[pl.ds(p * 256, 256)], usem)

        def iwait():
            pltpu.make_async_copy(idx_hbm.at[pl.ds(0, 2)],
                                  idx_v.at[pl.ds(0, 2)], isem).wait()

        def uwait():
            pltpu.make_async_copy(u_hbm.at[pl.ds(0, 256)],
                                  u_v.at[pl.ds(0, 256)], usem).wait()

        def swait():
            pltpu.make_async_copy(u_hbm.at[pl.ds(0, 128)],
                                  u_v.at[pl.ds(0, 128)], ssem).wait()

        def sub_window(w, p, cbase, drain_pred):
            if drain_pred is None:
                swait()
                swait()
            else:
                @pl.when(drain_pred)
                def _():
                    swait()
                    swait()

            @pl.when((w + 1) * 16 + sid < nwin)
            def _():
                fire_loads((w + 1) * 16 + sid, 1 - p)
            iwait()
            uwait()
            for j in range(2):
                for t in range(8):
                    v = idx_v[p * 2 + j, pl.ds(t * 16, 16)]
                    lv = v - cbase
                    ok = (lv >= 0) & (lv < ch)
                    lidx_v[p * 2 + j, pl.ds(t * 16, 16)] = jnp.where(
                        ok, lv, dump)
            for j in range(2):
                pltpu.async_copy(u_v.at[pl.ds(p * 256 + j * 128, 128)],
                                 acc.at[lidx_v.at[p * 2 + j]], ssem,
                                 add=True)

        for c in range(cpc):
            cbase = (2 * c + cid) * ch
            pltpu.sync_copy(z_hbm.at[pl.ds(sid * zr, zr)],
                            acc.at[pl.ds(sid * zr, zr)])

            @pl.when(sid < nwin)
            def _():
                fire_loads(sid, 0)
            plsc.subcore_barrier()

            def body(w2, carry):
                wa = 2 * w2

                @pl.when(wa * 16 + sid < nwin)
                def _():
                    sub_window(wa, 0, cbase, w2 >= 1)

                @pl.when((wa + 1) * 16 + sid < nwin)
                def _():
                    sub_window(wa + 1, 1, cbase, None)
                return carry

            lax.fori_loop(0, nwt2, body, 0)

            @pl.when(sid < nwin)
            def _():
                swait()
                swait()
            plsc.subcore_barrier()
            pltpu.sync_copy(acc.at[pl.ds(sid * cr, cr)],
                            out_hbm.at[pl.ds(cbase + sid * cr, cr)])
            plsc.subcore_barrier()

    return k(upd, idx2, zeros)


def _ln(o, g, b):
    mu = jnp.mean(o, axis=-1, keepdims=True)
    c = o - mu
    var = jnp.mean(c * c, axis=-1, keepdims=True)
    return c * lax.rsqrt(var + 1e-5) * g + b


def _full(shape):
    return pl.BlockSpec(shape, lambda i: (0, 0))


def _edge_mlp(e2d, xs_g, xr_g, p, ns, blk):
    ecnt, de = e2d.shape
    nb = -(-ecnt // blk)
    dh = p["W0"].shape[1]
    w4 = p["W0"].reshape(ns, 3, _D, dh)
    w0e = w4[:, 0].reshape(ns * _D, dh)
    w0s = w4[:, 1].reshape(ns * _D, dh)
    w0r = w4[:, 2].reshape(ns * _D, dh)

    def body(e_ref, xs_ref, xr_ref, w0e_r, w0s_r, w0r_r, b0_r, w1_r, b1_r,
             g_r, bb_r, eu_ref, out_ref):
        e = e_ref[...]
        e3 = e.reshape(blk, ns, _D)
        xs = xs_ref[...].reshape(blk, ns, _D)
        xr = xr_ref[...].reshape(blk, ns, _D)
        w0e_v = w0e_r[...]
        w0s_v = w0s_r[...]
        w0r_v = w0r_r[...]
        acc = jnp.zeros((blk, dh), jnp.float32)
        for t in range(ns):
            acc += jnp.dot(e3[:, t, :], w0e_v[t * _D:(t + 1) * _D],
                           preferred_element_type=jnp.float32)
            acc += jnp.dot(xs[:, t, :], w0s_v[t * _D:(t + 1) * _D],
                           preferred_element_type=jnp.float32)
            acc += jnp.dot(xr[:, t, :], w0r_v[t * _D:(t + 1) * _D],
                           preferred_element_type=jnp.float32)
        h = jax.nn.relu(acc + b0_r[...])
        o = jnp.dot(h, w1_r[...], preferred_element_type=jnp.float32) \
            + b1_r[...]
        y = _ln(o, g_r[...], bb_r[...])
        eu_ref[...] = y
        out_ref[...] = e + y

    eu, out = pl.pallas_call(
        body,
        grid=(nb,),
        in_specs=[
            pl.BlockSpec((blk, de), lambda i: (i, 0)),
            pl.BlockSpec((ns * blk, _D), lambda i: (i, 0)),
            pl.BlockSpec((ns * blk, _D), lambda i: (i, 0)),
            _full(w0e.shape), _full(w0s.shape), _full(w0r.shape),
            _full((1, dh)), _full(p["W1"].shape), _full((1, de)),
            _full((1, de)), _full((1, de)),
        ],
        out_specs=[
            pl.BlockSpec((blk, de), lambda i: (i, 0)),
            pl.BlockSpec((blk, de), lambda i: (i, 0)),
        ],
        out_shape=[
            jax.ShapeDtypeStruct((nb * blk, de), jnp.float32),
            jax.ShapeDtypeStruct((ecnt, de), jnp.float32),
        ],
    )(e2d, xs_g, xr_g, w0e, w0s, w0r, p["b0"].reshape(1, -1), p["W1"],
      p["b1"].reshape(1, -1), p["gamma"].reshape(1, -1),
      p["beta"].reshape(1, -1))
    return eu, out


def _node_mlp(x, aggrs, p, blk=1024):
    n = x.shape[0]
    nb = -(-n // blk)
    na = len(aggrs)
    dh = p["W0"].shape[1]
    wparts = [p["W0"][t * _D:(t + 1) * _D] for t in range(na + 1)]

    def body(*refs):
        x_ref = refs[0]
        a_refs = refs[1:1 + na]
        w_refs = refs[1 + na:2 + 2 * na]
        b0_r, w1_r, b1_r, g_r, bb_r, out_ref = refs[2 + 2 * na:]
        xv = x_ref[...]
        acc = jnp.dot(xv, w_refs[0][...], preferred_element_type=jnp.float32)
        for t in range(na):
            acc += jnp.dot(a_refs[t][...], w_refs[t + 1][...],
                           preferred_element_type=jnp.float32)
        h = jax.nn.relu(acc + b0_r[...])
        o = jnp.dot(h, w1_r[...], preferred_element_type=jnp.float32) \
            + b1_r[...]
        out_ref[...] = xv + _ln(o, g_r[...], bb_r[...])

    out = pl.pallas_call(
        body,
        grid=(nb,),
        in_specs=(
            [pl.BlockSpec((blk, _D), lambda i: (i, 0))] * (1 + na)
            + [_full((_D, dh))] * (1 + na)
            + [_full((1, dh)), _full(p["W1"].shape), _full((1, _D)),
               _full((1, _D)), _full((1, _D))]
        ),
        out_specs=pl.BlockSpec((blk, _D), lambda i: (i, 0)),
        out_shape=jax.ShapeDtypeStruct((n, _D), jnp.float32),
    )(x, *aggrs, *wparts, p["b0"].reshape(1, -1), p["W1"],
      p["b1"].reshape(1, -1), p["gamma"].reshape(1, -1),
      p["beta"].reshape(1, -1))
    return out


def kernel(mesh_n, obj_n, mm_index, mo_index, om_index, ff_index,
           e_mm, e_mo, e_om, e_ff, params):
    s, r = 0, 1
    nm = mesh_n.shape[0]
    no = obj_n.shape[0]
    eff = e_ff.shape[0]

    def gidx(a):
        return _pad_idx(a.astype(jnp.int32), 512, 0)

    _DBG_GATHER = False
    _DBG_SCATTER = False
    if _DBG_GATHER:
        def _gath(table, idx):
            return jnp.take(table, idx, axis=0)
    else:
        _gath = _sc_gather
    if _DBG_SCATTER:
        def _scat(upd, idx, n_out, ch):
            n_chunks = -(-n_out // ch)
            return jax.ops.segment_sum(
                upd, jnp.minimum(idx, n_chunks * ch - 1),
                num_segments=n_chunks * ch)
    else:
        _scat = _sc_scatter_add

    mm_s = _gath(mesh_n, gidx(mm_index[s]))
    mm_r = _gath(mesh_n, gidx(mm_index[r]))
    mo_s = _gath(mesh_n, gidx(mo_index[s]))
    mo_r = _gath(obj_n, gidx(mo_index[r]))
    om_s = _gath(obj_n, gidx(om_index[s]))
    om_r = _gath(mesh_n, gidx(om_index[r]))
    ff_s = _gath(mesh_n, gidx(ff_index[s].reshape(-1)))
    ff_r = _gath(mesh_n, gidx(ff_index[r].reshape(-1)))

    eu_mm, out_mm = _edge_mlp(e_mm, mm_s, mm_r, params["mm"], 1, 1024)
    eu_mo, out_mo = _edge_mlp(e_mo, mo_s, mo_r, params["mo"], 1, 1024)
    eu_om, out_om = _edge_mlp(e_om, om_s, om_r, params["om"], 1, 1024)
    eu_ff, out_ff = _edge_mlp(e_ff.reshape(eff, 3 * _D), ff_s, ff_r,
                              params["ff"], 3, 512)

    def sidx(a, ep):
        a = a.astype(jnp.int32)
        return jnp.concatenate(
            [a, jnp.full((ep - a.shape[0],), _SENTINEL, jnp.int32)])

    aggr_mm = _scat(eu_mm, sidx(mm_index[r], eu_mm.shape[0]), nm, 6400)
    aggr_mo = _scat(eu_mo, sidx(mo_index[r], eu_mo.shape[0]), no, 5120)
    aggr_om = _scat(eu_om, sidx(om_index[r], eu_om.shape[0]), nm, 6400)
    eu_ff_rows = eu_ff.reshape(-1, _D)
    aggr_ff = _scat(eu_ff_rows,
                    sidx(ff_index[r].reshape(-1), eu_ff_rows.shape[0]),
                    nm, 6400)

    obj_out = _node_mlp(obj_n, [aggr_mo], params["obj"])
    mesh_out = _node_mlp(mesh_n, [aggr_om, aggr_mm, aggr_ff], params["mesh"])

    return (mesh_out, obj_out, out_mm, out_mo, out_om,
            out_ff.reshape(eff, 3, _D))


# speedup vs baseline: 4.7448x; 1.0367x over previous
import functools

import jax
import jax.numpy as jnp
from jax import lax
from jax.experimental import pallas as pl
from jax.experimental.pallas import tpu as pltpu
from jax.experimental.pallas import tpu_sc as plsc

_D = 128
_SENTINEL = 1 << 28


def _pad_idx(idx, mult, fill):
    e = idx.shape[0]
    ep = -(-e // mult) * mult
    if ep == e:
        return idx
    return jnp.concatenate([idx, jnp.full((ep - e,), fill, jnp.int32)])


def _sc_gather(table, idx):
    ep = idx.shape[0]
    nwin = ep // 256
    nwt = -(-nwin // 32)
    nwt2 = -(-nwt // 2)
    idx2 = idx.reshape(ep // 128, 128)
    mesh = plsc.VectorSubcoreMesh(core_axis_name="c", subcore_axis_name="s")

    @functools.partial(
        pl.kernel, mesh=mesh,
        out_type=jax.ShapeDtypeStruct((ep, _D), jnp.float32),
        scratch_types=[
            pltpu.VMEM((4, 128), jnp.int32),
            pltpu.VMEM((512, _D), jnp.float32),
            pltpu.SemaphoreType.DMA,
            pltpu.SemaphoreType.DMA,
            pltpu.SemaphoreType.DMA,
        ])
    def k(tab, ih, out, idx_v, rows_v, isem, gsem, osem):
        wid = lax.axis_index("s") * 2 + lax.axis_index("c")

        def iwait():
            pltpu.make_async_copy(ih.at[pl.ds(0, 2)],
                                  idx_v.at[pl.ds(0, 2)], isem).wait()

        def owait():
            pltpu.make_async_copy(tab.at[pl.ds(0, 256)],
                                  rows_v.at[pl.ds(0, 256)], osem).wait()

        def sub_window(w, p, drain_pred):
            if drain_pred is None:
                owait()
            else:
                @pl.when(drain_pred)
                def _():
                    owait()

            win = w * 32 + wid

            @pl.when(win + 32 < nwin)
            def _():
                pltpu.async_copy(ih.at[pl.ds((win + 32) * 2, 2)],
                                 idx_v.at[pl.ds((1 - p) * 2, 2)], isem)
            iwait()
            descs = [
                pltpu.async_copy(
                    tab.at[idx_v.at[p * 2 + j]],
                    rows_v.at[pl.ds(p * 256 + j * 128, 128)], gsem)
                for j in range(2)
            ]
            for d in descs:
                d.wait()
            pltpu.async_copy(rows_v.at[pl.ds(p * 256, 256)],
                             out.at[pl.ds(win * 256, 256)], osem)

        @pl.when(wid < nwin)
        def _():
            pltpu.async_copy(ih.at[pl.ds(wid * 2, 2)],
                             idx_v.at[pl.ds(0, 2)], isem)

        def body(w2, carry):
            wa = 2 * w2

            @pl.when(wa * 32 + wid < nwin)
            def _():
                sub_window(wa, 0, w2 >= 1)

            @pl.when((wa + 1) * 32 + wid < nwin)
            def _():
                sub_window(wa + 1, 1, None)
            return carry

        lax.fori_loop(0, nwt2, body, 0)

        @pl.when(wid < nwin)
        def _():
            owait()

    return k(table, idx2)


def _sc_scatter_add(upd, idx, n_out, ch):
    ep = upd.shape[0]
    n_chunks = -(-n_out // ch)
    assert n_chunks % 2 == 0 and ep % 512 == 0
    cpc = n_chunks // 2
    spr = ch + 256
    npad = n_chunks * ch
    nwin = ep // 128
    nwt = -(-nwin // 16)
    nwt4 = -(-nwt // 4)
    zr = spr // 16
    cr = ch // 16
    idx2 = idx.reshape(ep // 128, 128)
    zeros = jnp.zeros((spr, _D), jnp.float32)
    mesh = plsc.VectorSubcoreMesh(core_axis_name="c", subcore_axis_name="s")

    @functools.partial(
        pl.kernel, mesh=mesh,
        out_type=jax.ShapeDtypeStruct((npad, _D), jnp.float32),
        scratch_types=[
            pltpu.VMEM((4, 128), jnp.int32),
            pltpu.VMEM((4, 128), jnp.int32),
            pltpu.VMEM((512, _D), jnp.float32),
            pltpu.VMEM_SHARED((spr, _D), jnp.float32),
            pltpu.SemaphoreType.DMA,
            pltpu.SemaphoreType.DMA,
            pltpu.SemaphoreType.DMA,
        ])
    def k(u_hbm, idx_hbm, z_hbm, out_hbm, idx_v, lidx_v, u_v, acc,
          isem, usem, ssem):
        cid = lax.axis_index("c")
        sid = lax.axis_index("s")
        lane = lax.iota(jnp.int32, 16)
        dump = jnp.int32(ch) + sid * 16 + lane

        def fire_loads(win, p):
            pltpu.async_copy(idx_hbm.at[pl.ds(win, 1)],
                             idx_v.at[pl.ds(p, 1)], isem)
            pltpu.async_copy(u_hbm.at[pl.ds(win * 128, 128)],
                             u_v.at[pl.ds(p * 128, 128)], usem)

        def iwait():
            pltpu.make_async_copy(idx_hbm.at[pl.ds(0, 1)],
                                  idx_v.at[pl.ds(0, 1)], isem).wait()

        def uwait():
            pltpu.make_async_copy(u_hbm.at[pl.ds(0, 128)],
                                  u_v.at[pl.ds(0, 128)], usem).wait()

        def swait():
            pltpu.make_async_copy(u_hbm.at[pl.ds(0, 128)],
                                  u_v.at[pl.ds(0, 128)], ssem).wait()

        def sub_window(w, p, cbase, drain_pred):
            if drain_pred is None:
                swait()
            else:
                @pl.when(drain_pred)
                def _():
                    swait()

            @pl.when((w + 2) * 16 + sid < nwin)
            def _():
                fire_loads((w + 2) * 16 + sid, (p + 2) & 3)
            iwait()
            uwait()
            for t in range(8):
                v = idx_v[p, pl.ds(t * 16, 16)]
                lv = v - cbase
                ok = (lv >= 0) & (lv < ch)
                lidx_v[p, pl.ds(t * 16, 16)] = jnp.where(ok, lv, dump)
            pltpu.async_copy(u_v.at[pl.ds(p * 128, 128)],
                             acc.at[lidx_v.at[p]], ssem, add=True)

        for c in range(cpc):
            cbase = (2 * c + cid) * ch
            pltpu.sync_copy(z_hbm.at[pl.ds(sid * zr, zr)],
                            acc.at[pl.ds(sid * zr, zr)])

            @pl.when(sid < nwin)
            def _():
                fire_loads(sid, 0)

            @pl.when(16 + sid < nwin)
            def _():
                fire_loads(16 + sid, 1)
            plsc.subcore_barrier()

            def body(w4, carry):
                wa = 4 * w4
                for q in range(4):
                    @pl.when((wa + q) * 16 + sid < nwin)
                    def _(q=q):
                        sub_window(wa + q, q, cbase,
                                   None if q >= 2 else w4 >= 1)
                return carry

            lax.fori_loop(0, nwt4, body, 0)

            @pl.when(sid < nwin)
            def _():
                swait()

            @pl.when(16 + sid < nwin)
            def _():
                swait()
            plsc.subcore_barrier()
            pltpu.sync_copy(acc.at[pl.ds(sid * cr, cr)],
                            out_hbm.at[pl.ds(cbase + sid * cr, cr)])
            plsc.subcore_barrier()

    return k(upd, idx2, zeros)


def _ln(o, g, b):
    mu = jnp.mean(o, axis=-1, keepdims=True)
    c = o - mu
    var = jnp.mean(c * c, axis=-1, keepdims=True)
    return c * lax.rsqrt(var + 1e-5) * g + b


def _full(shape):
    return pl.BlockSpec(shape, lambda i: (0, 0))


def _edge_mlp(e2d, xs_g, xr_g, p, ns, blk):
    ecnt, de = e2d.shape
    nb = -(-ecnt // blk)
    dh = p["W0"].shape[1]
    w4 = p["W0"].reshape(ns, 3, _D, dh)
    w0e = w4[:, 0].reshape(ns * _D, dh)
    w0s = w4[:, 1].reshape(ns * _D, dh)
    w0r = w4[:, 2].reshape(ns * _D, dh)

    def body(e_ref, xs_ref, xr_ref, w0e_r, w0s_r, w0r_r, b0_r, w1_r, b1_r,
             g_r, bb_r, eu_ref, out_ref):
        e = e_ref[...]
        e3 = e.reshape(blk, ns, _D)
        xs = xs_ref[...].reshape(blk, ns, _D)
        xr = xr_ref[...].reshape(blk, ns, _D)
        w0e_v = w0e_r[...]
        w0s_v = w0s_r[...]
        w0r_v = w0r_r[...]
        acc = jnp.zeros((blk, dh), jnp.float32)
        for t in range(ns):
            acc += jnp.dot(e3[:, t, :], w0e_v[t * _D:(t + 1) * _D],
                           preferred_element_type=jnp.float32)
            acc += jnp.dot(xs[:, t, :], w0s_v[t * _D:(t + 1) * _D],
                           preferred_element_type=jnp.float32)
            acc += jnp.dot(xr[:, t, :], w0r_v[t * _D:(t + 1) * _D],
                           preferred_element_type=jnp.float32)
        h = jax.nn.relu(acc + b0_r[...])
        o = jnp.dot(h, w1_r[...], preferred_element_type=jnp.float32) \
            + b1_r[...]
        y = _ln(o, g_r[...], bb_r[...])
        eu_ref[...] = y
        out_ref[...] = e + y

    eu, out = pl.pallas_call(
        body,
        grid=(nb,),
        in_specs=[
            pl.BlockSpec((blk, de), lambda i: (i, 0)),
            pl.BlockSpec((ns * blk, _D), lambda i: (i, 0)),
            pl.BlockSpec((ns * blk, _D), lambda i: (i, 0)),
            _full(w0e.shape), _full(w0s.shape), _full(w0r.shape),
            _full((1, dh)), _full(p["W1"].shape), _full((1, de)),
            _full((1, de)), _full((1, de)),
        ],
        out_specs=[
            pl.BlockSpec((blk, de), lambda i: (i, 0)),
            pl.BlockSpec((blk, de), lambda i: (i, 0)),
        ],
        out_shape=[
            jax.ShapeDtypeStruct((nb * blk, de), jnp.float32),
            jax.ShapeDtypeStruct((ecnt, de), jnp.float32),
        ],
    )(e2d, xs_g, xr_g, w0e, w0s, w0r, p["b0"].reshape(1, -1), p["W1"],
      p["b1"].reshape(1, -1), p["gamma"].reshape(1, -1),
      p["beta"].reshape(1, -1))
    return eu, out


def _node_mlp(x, aggrs, p, blk=1024):
    n = x.shape[0]
    nb = -(-n // blk)
    na = len(aggrs)
    dh = p["W0"].shape[1]
    wparts = [p["W0"][t * _D:(t + 1) * _D] for t in range(na + 1)]

    def body(*refs):
        x_ref = refs[0]
        a_refs = refs[1:1 + na]
        w_refs = refs[1 + na:2 + 2 * na]
        b0_r, w1_r, b1_r, g_r, bb_r, out_ref = refs[2 + 2 * na:]
        xv = x_ref[...]
        acc = jnp.dot(xv, w_refs[0][...], preferred_element_type=jnp.float32)
        for t in range(na):
            acc += jnp.dot(a_refs[t][...], w_refs[t + 1][...],
                           preferred_element_type=jnp.float32)
        h = jax.nn.relu(acc + b0_r[...])
        o = jnp.dot(h, w1_r[...], preferred_element_type=jnp.float32) \
            + b1_r[...]
        out_ref[...] = xv + _ln(o, g_r[...], bb_r[...])

    out = pl.pallas_call(
        body,
        grid=(nb,),
        in_specs=(
            [pl.BlockSpec((blk, _D), lambda i: (i, 0))] * (1 + na)
            + [_full((_D, dh))] * (1 + na)
            + [_full((1, dh)), _full(p["W1"].shape), _full((1, _D)),
               _full((1, _D)), _full((1, _D))]
        ),
        out_specs=pl.BlockSpec((blk, _D), lambda i: (i, 0)),
        out_shape=jax.ShapeDtypeStruct((n, _D), jnp.float32),
    )(x, *aggrs, *wparts, p["b0"].reshape(1, -1), p["W1"],
      p["b1"].reshape(1, -1), p["gamma"].reshape(1, -1),
      p["beta"].reshape(1, -1))
    return out


def kernel(mesh_n, obj_n, mm_index, mo_index, om_index, ff_index,
           e_mm, e_mo, e_om, e_ff, params):
    s, r = 0, 1
    nm = mesh_n.shape[0]
    no = obj_n.shape[0]
    eff = e_ff.shape[0]

    def gidx(a):
        return _pad_idx(a.astype(jnp.int32), 512, 0)

    _DBG_GATHER = False
    _DBG_SCATTER = False
    if _DBG_GATHER:
        def _gath(table, idx):
            return jnp.take(table, idx, axis=0)
    else:
        _gath = _sc_gather
    if _DBG_SCATTER:
        def _scat(upd, idx, n_out, ch):
            n_chunks = -(-n_out // ch)
            return jax.ops.segment_sum(
                upd, jnp.minimum(idx, n_chunks * ch - 1),
                num_segments=n_chunks * ch)
    else:
        _scat = _sc_scatter_add

    mm_s = _gath(mesh_n, gidx(mm_index[s]))
    mm_r = _gath(mesh_n, gidx(mm_index[r]))
    mo_s = _gath(mesh_n, gidx(mo_index[s]))
    mo_r = _gath(obj_n, gidx(mo_index[r]))
    om_s = _gath(obj_n, gidx(om_index[s]))
    om_r = _gath(mesh_n, gidx(om_index[r]))
    ff_s = _gath(mesh_n, gidx(ff_index[s].reshape(-1)))
    ff_r = _gath(mesh_n, gidx(ff_index[r].reshape(-1)))

    eu_mm, out_mm = _edge_mlp(e_mm, mm_s, mm_r, params["mm"], 1, 1024)
    eu_mo, out_mo = _edge_mlp(e_mo, mo_s, mo_r, params["mo"], 1, 1024)
    eu_om, out_om = _edge_mlp(e_om, om_s, om_r, params["om"], 1, 1024)
    eu_ff, out_ff = _edge_mlp(e_ff.reshape(eff, 3 * _D), ff_s, ff_r,
                              params["ff"], 3, 512)

    def sidx(a, ep):
        a = a.astype(jnp.int32)
        return jnp.concatenate(
            [a, jnp.full((ep - a.shape[0],), _SENTINEL, jnp.int32)])

    aggr_mm = _scat(eu_mm, sidx(mm_index[r], eu_mm.shape[0]), nm, 6400)
    aggr_mo = _scat(eu_mo, sidx(mo_index[r], eu_mo.shape[0]), no, 5120)
    aggr_om = _scat(eu_om, sidx(om_index[r], eu_om.shape[0]), nm, 6400)
    eu_ff_rows = eu_ff.reshape(-1, _D)
    aggr_ff = _scat(eu_ff_rows,
                    sidx(ff_index[r].reshape(-1), eu_ff_rows.shape[0]),
                    nm, 6400)

    obj_out = _node_mlp(obj_n, [aggr_mo], params["obj"])
    mesh_out = _node_mlp(mesh_n, [aggr_om, aggr_mm, aggr_ff], params["mesh"])

    return (mesh_out, obj_out, out_mm, out_mo, out_om,
            out_ff.reshape(eff, 3, _D))
